# baseline (speedup 1.0000x reference)
import jax
import jax.numpy as jnp
from jax import lax
from jax.experimental import pallas as pl
from jax.experimental.pallas import tpu as pltpu
from jax.experimental.pallas import tpu_sc as plsc

N_FIELDS = 26
VOCAB = 100000
N_EMBD = 32
BATCH = 4096

NC = 2
NS = 16
NW = NC * NS
BPW = BATCH // NW
LANES = 16

PACK = 128 // N_EMBD
GROUPS = VOCAB // PACK
CH = 2
NCHUNK = N_FIELDS // CH


def _body(gq_hbm, sub_hbm, tq_hbm, out_hbm, gq_v, sub_v, rows_v, acc_v,
          sem0, sem1):
    cid = lax.axis_index("c")
    sid = lax.axis_index("s")
    wid = sid * NC + cid
    base = wid * BPW

    pltpu.sync_copy(gq_hbm.at[:, pl.ds(base, BPW)], gq_v)
    pltpu.sync_copy(sub_hbm.at[:, pl.ds(base, BPW)], sub_v)

    sems = (sem0, sem1)

    def fire(c, buf):
        cps = []
        for fi in range(CH):
            f = c * CH + fi
            cp = pltpu.make_async_copy(
                tq_hbm.at[f].at[gq_v.at[f]], rows_v.at[buf, fi], sems[buf]
            )
            cp.start()
            cps.append(cp)
        return cps

    pending = fire(0, 0)
    for c in range(NCHUNK):
        nxt = None
        if c + 1 < NCHUNK:
            nxt = fire(c + 1, (c + 1) % 2)
        for cp in pending:
            cp.wait()
        pending = nxt

        bb = c % 2
        f0 = c * CH
        first = c == 0

        def reduce_block(blk, carry):
            j0 = blk * LANES
            jvec = lax.iota(jnp.int32, LANES) + j0
            svs = [sub_v[f0 + fi, pl.ds(j0, LANES)] for fi in range(CH)]
            for d in range(2 * LANES):
                cols = [sv + d for sv in svs]
                vals0 = plsc.load_gather(rows_v.at[bb, 0], [jvec, cols[0]])
                vals1 = plsc.load_gather(rows_v.at[bb, 1], [jvec, cols[1]])
                a = vals0 + vals1
                if not first:
                    a = a + acc_v[d, pl.ds(j0, LANES)]
                acc_v[d, pl.ds(j0, LANES)] = a
            return carry

        lax.fori_loop(0, BPW // LANES, reduce_block, 0)

    pltpu.sync_copy(acc_v, out_hbm.at[:, pl.ds(base, BPW)])


@jax.jit
def kernel(latents, tables):
    idx = latents.astype(jnp.int32).T
    gq = idx >> 2
    sub = (idx & 3) << 5
    tq = tables.reshape(N_FIELDS, GROUPS, PACK * N_EMBD)

    mesh = plsc.VectorSubcoreMesh(
        core_axis_name="c", subcore_axis_name="s", num_cores=NC, num_subcores=NS
    )
    run = pl.kernel(
        _body,
        out_type=jax.ShapeDtypeStruct((N_EMBD, BATCH), jnp.float32),
        mesh=mesh,
        scratch_types=[
            pltpu.VMEM((N_FIELDS, BPW), jnp.int32),
            pltpu.VMEM((N_FIELDS, BPW), jnp.int32),
            pltpu.VMEM((2, CH, BPW, PACK * N_EMBD), jnp.float32),
            pltpu.VMEM((N_EMBD, BPW), jnp.float32),
            pltpu.SemaphoreType.DMA,
            pltpu.SemaphoreType.DMA,
        ],
        compiler_params=pltpu.CompilerParams(
            use_tc_tiling_on_sc=True, needs_layout_passes=False
        ),
    )
    out_t = run(gq, sub, tq)
    return out_t.T


# speedup vs baseline: 1.0618x; 1.0618x over previous
import jax
import jax.numpy as jnp
from jax import lax
from jax.experimental import pallas as pl
from jax.experimental.pallas import tpu as pltpu
from jax.experimental.pallas import tpu_sc as plsc

N_FIELDS = 26
VOCAB = 100000
N_EMBD = 32
BATCH = 4096

NC = 2
NS = 16
NW = NC * NS
BPW = BATCH // NW
LANES = 16


def _body(idx_hbm, table_hbm, out_hbm, idx_v, rows_v, out_v, sem):
    cid = lax.axis_index("c")
    sid = lax.axis_index("s")
    wid = sid * NC + cid
    base = wid * BPW

    pltpu.sync_copy(idx_hbm.at[:, pl.ds(base, BPW)], idx_v)

    copies = []
    for f in range(N_FIELDS):
        cp = pltpu.make_async_copy(
            table_hbm.at[f].at[idx_v.at[f]], rows_v.at[f], sem
        )
        cp.start()
        copies.append(cp)
    for cp in copies:
        cp.wait()

    def body_j(j, carry):
        for d in (0, LANES):
            acc = rows_v[0, j, pl.ds(d, LANES)]
            for f in range(1, N_FIELDS):
                acc = acc + rows_v[f, j, pl.ds(d, LANES)]
            out_v[j, pl.ds(d, LANES)] = acc
        return carry

    lax.fori_loop(0, BPW, body_j, 0)

    pltpu.sync_copy(out_v, out_hbm.at[pl.ds(base, BPW)])


@jax.jit
def kernel(latents, tables):
    idx = latents.astype(jnp.int32).T

    mesh = plsc.VectorSubcoreMesh(
        core_axis_name="c", subcore_axis_name="s", num_cores=NC, num_subcores=NS
    )
    run = pl.kernel(
        _body,
        out_type=jax.ShapeDtypeStruct((BATCH, N_EMBD), jnp.float32),
        mesh=mesh,
        scratch_types=[
            pltpu.VMEM((N_FIELDS, BPW), jnp.int32),
            pltpu.VMEM((N_FIELDS, BPW, N_EMBD), jnp.float32),
            pltpu.VMEM((BPW, N_EMBD), jnp.float32),
            pltpu.SemaphoreType.DMA,
        ],
        compiler_params=pltpu.CompilerParams(use_tc_tiling_on_sc=False),
    )
    return run(idx, tables)
